# baseline (speedup 1.0000x reference)
import functools

import jax
import jax.numpy as jnp
from jax.experimental import pallas as pl
from jax.experimental.pallas import tpu as pltpu

_EPAD = 128


def _router_kernel(x_ref, lng_ref, lnb_ref, W1_ref, b1_ref, W2_ref, b2_ref,
                   idx_ref, probs_ref, aux_ref, acc_ref, *, n_tokens, n_blocks,
                   n_experts):
    i = pl.program_id(0)

    @pl.when(i == 0)
    def _init():
        acc_ref[...] = jnp.zeros_like(acc_ref)

    T = x_ref.shape[0]
    CH = min(128, T)
    W1 = W1_ref[...]
    W2 = W2_ref[...]
    b1 = b1_ref[...]
    b2 = b2_ref[...]
    lng = lng_ref[...]
    lnb = lnb_ref[...]
    acc = jnp.zeros_like(acc_ref)
    for c in range(T // CH):
        sl = pl.ds(c * CH, CH)
        xb = x_ref[sl, :]
        mu = jnp.mean(xb, axis=1, keepdims=True)
        xc = xb - mu
        var = jnp.mean(xc * xc, axis=1, keepdims=True)
        xn = xc / jnp.sqrt(var + 1e-5) * lng + lnb

        h = jnp.dot(xn, W1, preferred_element_type=jnp.float32) + b1
        h = jnp.maximum(h, 0.0)
        logits = jnp.dot(h, W2, preferred_element_type=jnp.float32) + b2

        m = jnp.max(logits, axis=1, keepdims=True)
        e = jnp.exp(logits - m)
        denom = jnp.sum(e, axis=1, keepdims=True)
        probs = e / denom

        acc = acc + jnp.sum(probs, axis=0, keepdims=True)

        iota = jax.lax.broadcasted_iota(jnp.int32, probs.shape, 1)
        big = jnp.int32(2 ** 30)
        p1 = jnp.max(probs, axis=1, keepdims=True)
        i1 = jnp.min(jnp.where(probs == p1, iota, big), axis=1, keepdims=True)
        pm = jnp.where(iota == i1, -1.0, probs)
        p2 = jnp.max(pm, axis=1, keepdims=True)
        i2 = jnp.min(jnp.where(pm == p2, iota, big), axis=1, keepdims=True)
        s = p1 + p2

        idx_ref[sl, :] = jnp.where(iota == 0, i1, i2).astype(jnp.int32)
        probs_ref[sl, :] = jnp.where(iota == 0, p1 / s, p2 / s)

    acc_ref[...] += acc

    @pl.when(i == n_blocks - 1)
    def _finalize():
        rp = acc_ref[...] / jnp.float32(n_tokens)
        aux = jnp.sum(rp * jnp.log(rp * jnp.float32(n_experts) + 1e-9),
                      axis=1, keepdims=True)
        aux_ref[...] = aux


def kernel(x, ln_g, ln_b, W1, b1, W2, b2):
    B, S, H = x.shape
    E = W2.shape[1]
    N = B * S
    T = min(512, N)
    n_blocks = N // T

    xf = x.reshape(N, H)
    lng = ln_g.reshape(1, H)
    lnb = ln_b.reshape(1, H)
    b1r = b1.reshape(1, H)
    W2p = jnp.zeros((H, _EPAD), W2.dtype).at[:, :E].set(W2)
    b2p = jnp.full((1, _EPAD), -1e30, b2.dtype).at[0, :E].set(b2)

    grid = (n_blocks,)
    kern = functools.partial(_router_kernel, n_tokens=N, n_blocks=n_blocks,
                             n_experts=E)
    idx, probs, aux = pl.pallas_call(
        kern,
        grid=grid,
        in_specs=[
            pl.BlockSpec((T, H), lambda i: (i, 0)),
            pl.BlockSpec((1, H), lambda i: (0, 0)),
            pl.BlockSpec((1, H), lambda i: (0, 0)),
            pl.BlockSpec((H, H), lambda i: (0, 0)),
            pl.BlockSpec((1, H), lambda i: (0, 0)),
            pl.BlockSpec((H, _EPAD), lambda i: (0, 0)),
            pl.BlockSpec((1, _EPAD), lambda i: (0, 0)),
        ],
        out_specs=[
            pl.BlockSpec((T, _EPAD), lambda i: (i, 0)),
            pl.BlockSpec((T, _EPAD), lambda i: (i, 0)),
            pl.BlockSpec((1, 1), lambda i: (0, 0)),
        ],
        out_shape=[
            jax.ShapeDtypeStruct((N, _EPAD), jnp.int32),
            jax.ShapeDtypeStruct((N, _EPAD), jnp.float32),
            jax.ShapeDtypeStruct((1, 1), jnp.float32),
        ],
        scratch_shapes=[pltpu.VMEM((1, _EPAD), jnp.float32)],
        compiler_params=pltpu.CompilerParams(
            dimension_semantics=("arbitrary",),
        ),
    )(xf, lng, lnb, W1, b1r, W2p, b2p)

    top_k_indices = idx[:, :2].reshape(B, S, 2)
    top_k_probs = probs[:, :2].reshape(B, S, 2)
    aux_loss = aux[0, 0]
    return (top_k_indices, top_k_probs, aux_loss)


# speedup vs baseline: 1.0842x; 1.0842x over previous
import functools

import jax
import jax.numpy as jnp
from jax.experimental import pallas as pl
from jax.experimental.pallas import tpu as pltpu

_EPAD = 128


def _router_kernel(x_ref, lng_ref, lnb_ref, W1_ref, b1_ref, W2_ref, b2_ref,
                   idx_ref, probs_ref, aux_ref, acc_ref, *, n_tokens, n_blocks,
                   n_experts):
    i = pl.program_id(0)

    @pl.when(i == 0)
    def _init():
        acc_ref[...] = jnp.zeros_like(acc_ref)

    xb = x_ref[...]
    mu = jnp.mean(xb, axis=1, keepdims=True)
    xc = xb - mu
    var = jnp.mean(xc * xc, axis=1, keepdims=True)
    xn = xc / jnp.sqrt(var + 1e-5) * lng_ref[...] + lnb_ref[...]

    h = jnp.dot(xn, W1_ref[...], preferred_element_type=jnp.float32) + b1_ref[...]
    h = jnp.maximum(h, 0.0)
    logits = jnp.dot(h, W2_ref[...], preferred_element_type=jnp.float32) + b2_ref[...]

    m = jnp.max(logits, axis=1, keepdims=True)
    e = jnp.exp(logits - m)
    denom = jnp.sum(e, axis=1, keepdims=True)
    probs = e / denom

    acc_ref[...] += jnp.sum(probs, axis=0, keepdims=True)

    iota = jax.lax.broadcasted_iota(jnp.int32, probs.shape, 1)
    big = jnp.int32(2 ** 30)
    p1 = jnp.max(probs, axis=1, keepdims=True)
    i1 = jnp.min(jnp.where(probs == p1, iota, big), axis=1, keepdims=True)
    pm = jnp.where(iota == i1, -1.0, probs)
    p2 = jnp.max(pm, axis=1, keepdims=True)
    i2 = jnp.min(jnp.where(pm == p2, iota, big), axis=1, keepdims=True)
    s = p1 + p2

    idx_ref[...] = jnp.where(iota == 0, i1, i2).astype(jnp.int32)
    probs_ref[...] = jnp.where(iota == 0, p1 / s, p2 / s)

    @pl.when(i == n_blocks - 1)
    def _finalize():
        rp = acc_ref[...] / jnp.float32(n_tokens)
        aux = jnp.sum(rp * jnp.log(rp * jnp.float32(n_experts) + 1e-9),
                      axis=1, keepdims=True)
        aux_ref[...] = aux


def kernel(x, ln_g, ln_b, W1, b1, W2, b2):
    B, S, H = x.shape
    E = W2.shape[1]
    N = B * S
    T = min(1024, N)
    n_blocks = N // T

    xf = x.reshape(N, H)
    lng = ln_g.reshape(1, H)
    lnb = ln_b.reshape(1, H)
    b1r = b1.reshape(1, H)
    W2p = jnp.zeros((H, _EPAD), W2.dtype).at[:, :E].set(W2)
    b2p = jnp.full((1, _EPAD), -1e30, b2.dtype).at[0, :E].set(b2)

    grid = (n_blocks,)
    kern = functools.partial(_router_kernel, n_tokens=N, n_blocks=n_blocks,
                             n_experts=E)
    idx, probs, aux = pl.pallas_call(
        kern,
        grid=grid,
        in_specs=[
            pl.BlockSpec((T, H), lambda i: (i, 0)),
            pl.BlockSpec((1, H), lambda i: (0, 0)),
            pl.BlockSpec((1, H), lambda i: (0, 0)),
            pl.BlockSpec((H, H), lambda i: (0, 0)),
            pl.BlockSpec((1, H), lambda i: (0, 0)),
            pl.BlockSpec((H, _EPAD), lambda i: (0, 0)),
            pl.BlockSpec((1, _EPAD), lambda i: (0, 0)),
        ],
        out_specs=[
            pl.BlockSpec((T, _EPAD), lambda i: (i, 0)),
            pl.BlockSpec((T, _EPAD), lambda i: (i, 0)),
            pl.BlockSpec((1, 1), lambda i: (0, 0)),
        ],
        out_shape=[
            jax.ShapeDtypeStruct((N, _EPAD), jnp.int32),
            jax.ShapeDtypeStruct((N, _EPAD), jnp.float32),
            jax.ShapeDtypeStruct((1, 1), jnp.float32),
        ],
        scratch_shapes=[pltpu.VMEM((1, _EPAD), jnp.float32)],
        compiler_params=pltpu.CompilerParams(
            dimension_semantics=("arbitrary",),
        ),
    )(xf, lng, lnb, W1, b1r, W2p, b2p)

    top_k_indices = idx[:, :2].reshape(B, S, 2)
    top_k_probs = probs[:, :2].reshape(B, S, 2)
    aux_loss = aux[0, 0]
    return (top_k_indices, top_k_probs, aux_loss)


# speedup vs baseline: 1.0852x; 1.0010x over previous
import functools

import jax
import jax.numpy as jnp
from jax.experimental import pallas as pl
from jax.experimental.pallas import tpu as pltpu

_EPAD = 128


def _router_kernel(x_ref, lng_ref, lnb_ref, W1_ref, b1_ref, W2_ref, b2_ref,
                   idx_ref, probs_ref, aux_ref, acc_ref, *, n_tokens, n_blocks,
                   n_experts):
    i = pl.program_id(0)

    @pl.when(i == 0)
    def _init():
        acc_ref[...] = jnp.zeros_like(acc_ref)

    xb = x_ref[...]
    mu = jnp.mean(xb, axis=1, keepdims=True)
    xc = xb - mu
    var = jnp.mean(xc * xc, axis=1, keepdims=True)
    xn = xc / jnp.sqrt(var + 1e-5) * lng_ref[...] + lnb_ref[...]

    h = jnp.dot(xn, W1_ref[...], preferred_element_type=jnp.float32) + b1_ref[...]
    h = jnp.maximum(h, 0.0)
    logits = jnp.dot(h, W2_ref[...], preferred_element_type=jnp.float32) + b2_ref[...]

    m = jnp.max(logits, axis=1, keepdims=True)
    e = jnp.exp(logits - m)
    denom = jnp.sum(e, axis=1, keepdims=True)
    probs = e / denom

    acc_ref[...] += jnp.sum(probs, axis=0, keepdims=True)

    iota = jax.lax.broadcasted_iota(jnp.int32, probs.shape, 1)
    big = jnp.int32(2 ** 30)
    p1 = jnp.max(probs, axis=1, keepdims=True)
    i1 = jnp.min(jnp.where(probs == p1, iota, big), axis=1, keepdims=True)
    pm = jnp.where(iota == i1, -1.0, probs)
    p2 = jnp.max(pm, axis=1, keepdims=True)
    i2 = jnp.min(jnp.where(pm == p2, iota, big), axis=1, keepdims=True)
    s = p1 + p2

    idx_ref[...] = jnp.concatenate([i1, i2], axis=1)
    probs_ref[...] = jnp.concatenate([p1 / s, p2 / s], axis=1)

    @pl.when(i == n_blocks - 1)
    def _finalize():
        rp = acc_ref[...] / jnp.float32(n_tokens)
        aux = jnp.sum(rp * jnp.log(rp * jnp.float32(n_experts) + 1e-9),
                      axis=1, keepdims=True)
        aux_ref[...] = aux


def kernel(x, ln_g, ln_b, W1, b1, W2, b2):
    B, S, H = x.shape
    E = W2.shape[1]
    N = B * S
    T = min(1024, N)
    n_blocks = N // T

    xf = x.reshape(N, H)
    lng = ln_g.reshape(1, H)
    lnb = ln_b.reshape(1, H)
    b1r = b1.reshape(1, H)
    W2p = jnp.zeros((H, _EPAD), W2.dtype).at[:, :E].set(W2)
    b2p = jnp.full((1, _EPAD), -1e30, b2.dtype).at[0, :E].set(b2)

    grid = (n_blocks,)
    kern = functools.partial(_router_kernel, n_tokens=N, n_blocks=n_blocks,
                             n_experts=E)
    idx, probs, aux = pl.pallas_call(
        kern,
        grid=grid,
        in_specs=[
            pl.BlockSpec((T, H), lambda i: (i, 0)),
            pl.BlockSpec((1, H), lambda i: (0, 0)),
            pl.BlockSpec((1, H), lambda i: (0, 0)),
            pl.BlockSpec((H, H), lambda i: (0, 0)),
            pl.BlockSpec((1, H), lambda i: (0, 0)),
            pl.BlockSpec((H, _EPAD), lambda i: (0, 0)),
            pl.BlockSpec((1, _EPAD), lambda i: (0, 0)),
        ],
        out_specs=[
            pl.BlockSpec((T, 2), lambda i: (i, 0)),
            pl.BlockSpec((T, 2), lambda i: (i, 0)),
            pl.BlockSpec((1, 1), lambda i: (0, 0)),
        ],
        out_shape=[
            jax.ShapeDtypeStruct((N, 2), jnp.int32),
            jax.ShapeDtypeStruct((N, 2), jnp.float32),
            jax.ShapeDtypeStruct((1, 1), jnp.float32),
        ],
        scratch_shapes=[pltpu.VMEM((1, _EPAD), jnp.float32)],
        compiler_params=pltpu.CompilerParams(
            dimension_semantics=("arbitrary",),
        ),
    )(xf, lng, lnb, W1, b1r, W2p, b2p)

    top_k_indices = idx.reshape(B, S, 2)
    top_k_probs = probs.reshape(B, S, 2)
    aux_loss = aux[0, 0]
    return (top_k_indices, top_k_probs, aux_loss)
